# Initial kernel scaffold; baseline (speedup 1.0000x reference)
import jax
import jax.numpy as jnp
from jax.experimental import pallas as pl


def kernel(combined_hm_preds, cur_batch, num_of_kp):
    raise NotImplementedError("write your pallas kernel here")


# speedup vs baseline: 1.0589x; 1.0589x over previous
import functools

import jax
import jax.numpy as jnp
from jax.experimental import pallas as pl
from jax.experimental.pallas import tpu as pltpu


def _kp_kernel(x_ref, map_ref, zeta_ref, kpx_ref, kpy_ref,
               zs_ref, xm_ref, ym_ref, *, hb_count, hb_size):
    hb = pl.program_id(1)
    x = x_ref[0]
    k_dim, hb_dim, w_dim = x.shape

    m = jnp.max(x, axis=0, keepdims=True)
    e = jnp.exp(x - m)
    s = jnp.sum(e, axis=0, keepdims=True)
    p = e * (1.0 / s)
    map_ref[0] = p

    y_off = (hb * hb_size).astype(jnp.float32)
    yw = jax.lax.broadcasted_iota(
        jnp.int32, (1, hb_dim, w_dim), 1).astype(jnp.float32) + y_off

    colsum = jnp.sum(p, axis=1)
    ycolsum = jnp.sum(p * yw, axis=1)
    xs = jax.lax.broadcasted_iota(jnp.int32, (1, w_dim), 1).astype(jnp.float32)

    @pl.when(hb == 0)
    def _init():
        zs_ref[...] = jnp.zeros_like(zs_ref)
        xm_ref[...] = jnp.zeros_like(xm_ref)
        ym_ref[...] = jnp.zeros_like(ym_ref)

    zs_ref[...] += colsum
    xm_ref[...] += colsum * xs
    ym_ref[...] += ycolsum

    @pl.when(hb == hb_count - 1)
    def _finalize():
        zeta = jnp.sum(zs_ref[...], axis=1)
        xmom = jnp.sum(xm_ref[...], axis=1)
        ymom = jnp.sum(ym_ref[...], axis=1)
        rz = 1.0 / zeta
        zeta_ref[0, 0, :] = zeta
        kpx_ref[0, 0, :] = jnp.round(xmom * rz)
        kpy_ref[0, 0, :] = jnp.round(ymom * rz)


def kernel(combined_hm_preds, cur_batch, num_of_kp):
    B, K, H, W = combined_hm_preds.shape
    HB_SIZE = 32
    HB_COUNT = H // HB_SIZE

    kfn = functools.partial(_kp_kernel, hb_count=HB_COUNT, hb_size=HB_SIZE)
    f32 = jnp.float32
    small = jax.ShapeDtypeStruct((B, 1, K), f32)
    map_out, zeta3, kpx3, kpy3 = pl.pallas_call(
        kfn,
        grid=(B, HB_COUNT),
        in_specs=[
            pl.BlockSpec((1, K, HB_SIZE, W), lambda b, hb: (b, 0, hb, 0)),
        ],
        out_specs=[
            pl.BlockSpec((1, K, HB_SIZE, W), lambda b, hb: (b, 0, hb, 0)),
            pl.BlockSpec((1, 1, K), lambda b, hb: (b, 0, 0)),
            pl.BlockSpec((1, 1, K), lambda b, hb: (b, 0, 0)),
            pl.BlockSpec((1, 1, K), lambda b, hb: (b, 0, 0)),
        ],
        out_shape=[
            jax.ShapeDtypeStruct((B, K, H, W), f32),
            small, small, small,
        ],
        scratch_shapes=[
            pltpu.VMEM((K, W), f32),
            pltpu.VMEM((K, W), f32),
            pltpu.VMEM((K, W), f32),
        ],
        compiler_params=pltpu.CompilerParams(
            dimension_semantics=("parallel", "arbitrary"),
        ),
    )(combined_hm_preds)

    zeta = zeta3[:, 0, :]
    keypoint = jnp.stack([kpx3[:, 0, :], kpy3[:, 0, :]], axis=-1)
    return (map_out, keypoint, zeta)
